# Initial kernel scaffold; baseline (speedup 1.0000x reference)
import jax
import jax.numpy as jnp
from jax.experimental import pallas as pl


def kernel(features, edge_index, W1, b1, W2, b2):
    raise NotImplementedError("write your pallas kernel here")


# speedup vs baseline: 17.7873x; 17.7873x over previous
import functools

import jax
import jax.numpy as jnp
from jax import lax
from jax.experimental import pallas as pl
from jax.experimental.pallas import tpu as pltpu
from jax.experimental.pallas import tpu_sc as plsc

N = 100000
E = 1600000
F_IN = 10
HID = 16
C_PAD = 16

LANES = 128
EROWS = E // LANES

NT = N // 16
ZROWS = 625

f32 = jnp.float32
i32 = jnp.int32

_MESH = plsc.VectorSubcoreMesh(core_axis_name="c", subcore_axis_name="s")
_SC_PARAMS = pltpu.CompilerParams(use_tc_tiling_on_sc=False)


def _fill(ref, rows, value):
    def body(i, _):
        ref[i, :] = jnp.full((16,), value, f32)
        return 0
    lax.fori_loop(0, rows, body, 0)


def _zero_acc_rows(zbuf, acc, s):
    def body(m, _):
        pltpu.sync_copy(zbuf, acc.at[pl.ds(s * NT + m * ZROWS, ZROWS)])
        return 0
    lax.fori_loop(0, NT // ZROWS, body, 0)


def _copy_out_rows(zbuf, acc, out_ref, s):
    def body(m, _):
        sl = pl.ds(s * NT + m * ZROWS, ZROWS)
        pltpu.sync_copy(acc.at[sl], zbuf)
        pltpu.sync_copy(zbuf, out_ref.at[sl])
        return 0
    lax.fori_loop(0, NT // ZROWS, body, 0)


@functools.partial(
    pl.kernel,
    out_type=[jax.ShapeDtypeStruct((N, 16), f32),
              jax.ShapeDtypeStruct((N, 16), f32)],
    mesh=_MESH,
    compiler_params=_SC_PARAMS,
    scratch_types=[
        pltpu.VMEM((ZROWS, 16), f32),
        pltpu.VMEM((LANES, 16), f32),
        pltpu.VMEM((65, LANES), i32),
        pltpu.VMEM((1, LANES), i32),
        pltpu.VMEM_SHARED((N, 16), f32),
    ],
)
def _deg_kernel(dst_hbm, dega, degb, zbuf, ones, idx, idxt, acc):
    c = lax.axis_index("c")
    s = lax.axis_index("s")
    _fill(zbuf, ZROWS, 0.0)
    _fill(ones, LANES, 1.0)
    _zero_acc_rows(zbuf, acc, s)
    plsc.subcore_barrier()

    def chunk(k, _):
        row0 = c * 6250 + s * 390 + k * 65
        pltpu.sync_copy(dst_hbm.at[pl.ds(row0, 65)], idx)

        def inner(j, _):
            pltpu.sync_copy(ones, acc.at[idx.at[j]], add=True)
            return 0
        lax.fori_loop(0, 65, inner, 0)
        return 0
    lax.fori_loop(0, 6, chunk, 0)

    @pl.when(s < 10)
    def _tail():
        pltpu.sync_copy(dst_hbm.at[pl.ds(c * 6250 + 6240 + s, 1)], idxt)
        pltpu.sync_copy(ones, acc.at[idxt.at[0]], add=True)

    plsc.subcore_barrier()

    @pl.when(c == 0)
    def _out0():
        _copy_out_rows(zbuf, acc, dega, s)

    @pl.when(c == 1)
    def _out1():
        _copy_out_rows(zbuf, acc, degb, s)


def _aggregate(src_hbm, dst_hbm, y_hbm, out_ref,
               sidx, didx, sidxt, didxt, rowbuf, acc, sem,
               s, row_base, rows_per_tile, kr, tail_rows):
    def chunk(k, _):
        row0 = row_base + s * rows_per_tile + k * kr
        pltpu.sync_copy(src_hbm.at[pl.ds(row0, kr)], sidx)
        pltpu.sync_copy(dst_hbm.at[pl.ds(row0, kr)], didx)

        def inner(j, _):
            pltpu.async_copy(y_hbm.at[sidx.at[j]], rowbuf, sem).wait()
            pltpu.sync_copy(rowbuf, acc.at[didx.at[j]], add=True)
            return 0
        lax.fori_loop(0, kr, inner, 0)
        return 0
    lax.fori_loop(0, rows_per_tile // kr, chunk, 0)

    @pl.when(s < tail_rows)
    def _tail():
        row = row_base + 16 * rows_per_tile + s
        pltpu.sync_copy(src_hbm.at[pl.ds(row, 1)], sidxt)
        pltpu.sync_copy(dst_hbm.at[pl.ds(row, 1)], didxt)
        pltpu.async_copy(y_hbm.at[sidxt.at[0]], rowbuf, sem).wait()
        pltpu.sync_copy(rowbuf, acc.at[didxt.at[0]], add=True)


@functools.partial(
    pl.kernel,
    out_type=[jax.ShapeDtypeStruct((N, 16), f32),
              jax.ShapeDtypeStruct((N, 16), f32)],
    mesh=_MESH,
    compiler_params=_SC_PARAMS,
    scratch_types=[
        pltpu.VMEM((ZROWS, 16), f32),
        pltpu.VMEM((71, LANES), i32),
        pltpu.VMEM((71, LANES), i32),
        pltpu.VMEM((1, LANES), i32),
        pltpu.VMEM((1, LANES), i32),
        pltpu.VMEM((LANES, 16), f32),
        pltpu.VMEM_SHARED((N, 16), f32),
        pltpu.SemaphoreType.DMA,
    ],
)
def _agg_feature_split(src_hbm, dst_hbm, ya_hbm, yb_hbm, agga, aggb,
                       zbuf, sidx, didx, sidxt, didxt, rowbuf, acc, sem):
    c = lax.axis_index("c")
    s = lax.axis_index("s")
    _fill(zbuf, ZROWS, 0.0)
    _zero_acc_rows(zbuf, acc, s)
    plsc.subcore_barrier()

    @pl.when(c == 0)
    def _run0():
        _aggregate(src_hbm, dst_hbm, ya_hbm, agga,
                   sidx, didx, sidxt, didxt, rowbuf, acc, sem,
                   s, 0, 781, 71, 4)

    @pl.when(c == 1)
    def _run1():
        _aggregate(src_hbm, dst_hbm, yb_hbm, aggb,
                   sidx, didx, sidxt, didxt, rowbuf, acc, sem,
                   s, 0, 781, 71, 4)

    plsc.subcore_barrier()

    @pl.when(c == 0)
    def _out0():
        _copy_out_rows(zbuf, acc, agga, s)

    @pl.when(c == 1)
    def _out1():
        _copy_out_rows(zbuf, acc, aggb, s)


@functools.partial(
    pl.kernel,
    out_type=[jax.ShapeDtypeStruct((N, 16), f32),
              jax.ShapeDtypeStruct((N, 16), f32)],
    mesh=_MESH,
    compiler_params=_SC_PARAMS,
    scratch_types=[
        pltpu.VMEM((ZROWS, 16), f32),
        pltpu.VMEM((65, LANES), i32),
        pltpu.VMEM((65, LANES), i32),
        pltpu.VMEM((1, LANES), i32),
        pltpu.VMEM((1, LANES), i32),
        pltpu.VMEM((LANES, 16), f32),
        pltpu.VMEM_SHARED((N, 16), f32),
        pltpu.SemaphoreType.DMA,
    ],
)
def _agg_edge_split(src_hbm, dst_hbm, y_hbm, agga, aggb,
                    zbuf, sidx, didx, sidxt, didxt, rowbuf, acc, sem):
    c = lax.axis_index("c")
    s = lax.axis_index("s")
    _fill(zbuf, ZROWS, 0.0)
    _zero_acc_rows(zbuf, acc, s)
    plsc.subcore_barrier()

    _aggregate(src_hbm, dst_hbm, y_hbm, None,
               sidx, didx, sidxt, didxt, rowbuf, acc, sem,
               s, c * 6250, 390, 65, 10)

    plsc.subcore_barrier()

    @pl.when(c == 0)
    def _out0():
        _copy_out_rows(zbuf, acc, agga, s)

    @pl.when(c == 1)
    def _out1():
        _copy_out_rows(zbuf, acc, aggb, s)


R = 2000


def _tc_a_body(dega, degb, x, w1, dinv16, y1a, y1b):
    deg = dega[:, :1] + degb[:, :1] + 1.0
    dinv = lax.rsqrt(deg)
    xw = jnp.dot(x[...], w1[...], preferred_element_type=f32)
    y = xw * dinv
    dinv16[...] = jnp.broadcast_to(dinv, (R, 16))
    y1a[...] = y[:, :16]
    y1b[...] = y[:, 16:]


def _tc_b_body(dinv16, agg1a, agg1b, y1a, y1b, w2, b1, y2):
    d = dinv16[...]
    h = jnp.concatenate([d * (agg1a[...] + y1a[...]),
                         d * (agg1b[...] + y1b[...])], axis=1)
    h = jnp.maximum(h + b1[...][None, :], 0.0)
    y2[...] = jnp.dot(h, w2[...], preferred_element_type=f32) * d


def _tc_c_body(dinv16, agg2a, agg2b, y2, b2, out):
    res = dinv16[...] * (agg2a[...] + agg2b[...] + y2[...]) + b2[...][None, :]
    out[...] = res[:, :10]


def _rb(cols):
    return pl.BlockSpec((R, cols), lambda i: (i, 0))


def _full(shape):
    nd = len(shape)
    return pl.BlockSpec(shape, lambda i: (0,) * nd)


def kernel(features, edge_index, W1, b1, W2, b2):
    src = edge_index[0].astype(i32).reshape(EROWS, LANES)
    dst = edge_index[1].astype(i32).reshape(EROWS, LANES)
    w2p = jnp.pad(W2, ((0, 0), (0, C_PAD - 10)))
    b2p = jnp.pad(b2, (0, C_PAD - 10))

    dega, degb = _deg_kernel(dst)

    dinv16, y1a, y1b = pl.pallas_call(
        _tc_a_body,
        grid=(N // R,),
        in_specs=[_rb(16), _rb(16), _rb(F_IN), _full((F_IN, 32))],
        out_specs=[_rb(16), _rb(16), _rb(16)],
        out_shape=[jax.ShapeDtypeStruct((N, 16), f32)] * 3,
    )(dega, degb, features, W1)

    agg1a, agg1b = _agg_feature_split(src, dst, y1a, y1b)

    y2 = pl.pallas_call(
        _tc_b_body,
        grid=(N // R,),
        in_specs=[_rb(16)] * 5 + [_full((32, C_PAD)), _full((32,))],
        out_specs=_rb(C_PAD),
        out_shape=jax.ShapeDtypeStruct((N, C_PAD), f32),
    )(dinv16, agg1a, agg1b, y1a, y1b, w2p, b1)

    agg2a, agg2b = _agg_edge_split(src, dst, y2)

    out = pl.pallas_call(
        _tc_c_body,
        grid=(N // R,),
        in_specs=[_rb(16)] * 4 + [_full((C_PAD,))],
        out_specs=_rb(10),
        out_shape=jax.ShapeDtypeStruct((N, 10), f32),
    )(dinv16, agg2a, agg2b, y2, b2p)

    return out


# speedup vs baseline: 27.9225x; 1.5698x over previous
import functools

import jax
import jax.numpy as jnp
from jax import lax
from jax.experimental import pallas as pl
from jax.experimental.pallas import tpu as pltpu
from jax.experimental.pallas import tpu_sc as plsc

N = 100000
E = 1600000
F_IN = 10
C_PAD = 16
NT = N // 16

f32 = jnp.float32
i32 = jnp.int32

_MESH = plsc.VectorSubcoreMesh(core_axis_name="c", subcore_axis_name="s")
_SC_PARAMS = pltpu.CompilerParams(use_tc_tiling_on_sc=False)


def _fill(ref, rows, value):
    def body(i, _):
        ref[i, :] = jnp.full((16,), value, f32)
        return 0
    lax.fori_loop(0, rows, body, 0)


def _stage_zero(buf, rows_buf, acc, s):
    full, rem = NT // rows_buf, NT % rows_buf

    def body(m, _):
        pltpu.sync_copy(buf, acc.at[pl.ds(s * NT + m * rows_buf, rows_buf)])
        return 0
    lax.fori_loop(0, full, body, 0)
    if rem:
        pltpu.sync_copy(buf.at[pl.ds(0, rem)],
                        acc.at[pl.ds(s * NT + full * rows_buf, rem)])


def _stage_out(buf, rows_buf, acc, out_ref, s):
    full, rem = NT // rows_buf, NT % rows_buf

    def body(m, _):
        sl = pl.ds(s * NT + m * rows_buf, rows_buf)
        pltpu.sync_copy(acc.at[sl], buf)
        pltpu.sync_copy(buf, out_ref.at[sl])
        return 0
    lax.fori_loop(0, full, body, 0)
    if rem:
        sl = pl.ds(s * NT + full * rows_buf, rem)
        bsl = buf.at[pl.ds(0, rem)]
        pltpu.sync_copy(acc.at[sl], bsl)
        pltpu.sync_copy(bsl, out_ref.at[sl])


CHD = 400


@functools.partial(
    pl.kernel,
    out_type=[jax.ShapeDtypeStruct((N, 16), f32),
              jax.ShapeDtypeStruct((N, 16), f32)],
    mesh=_MESH,
    compiler_params=_SC_PARAMS,
    scratch_types=[
        pltpu.VMEM((CHD, 16), f32),
        pltpu.VMEM((CHD, 16), f32),
        pltpu.VMEM((CHD,), i32),
        pltpu.VMEM((CHD,), i32),
        pltpu.VMEM_SHARED((N, 16), f32),
        pltpu.SemaphoreType.DMA,
        pltpu.SemaphoreType.DMA,
    ],
)
def _deg_kernel(dst_hbm, dega, degb, zbuf, ones, dxa, dxb, acc, sma, smb):
    c = lax.axis_index("c")
    s = lax.axis_index("s")
    _fill(zbuf, CHD, 0.0)
    _fill(ones, CHD, 1.0)
    _stage_zero(zbuf, CHD, acc, s)
    plsc.subcore_barrier()

    base = c * (E // 2) + s * (E // 32)
    nck = (E // 32) // CHD
    didx = (dxa, dxb)
    sems = (sma, smb)

    def load(k, p):
        pltpu.sync_copy(dst_hbm.at[pl.ds(base + k * CHD, CHD)], didx[p])

    def sissue(p):
        pltpu.async_copy(ones, acc.at[didx[p]], sems[p], add=True)

    def swait(p):
        pltpu.make_async_copy(ones, acc.at[didx[p]], sems[p]).wait()

    load(0, 0)
    sissue(0)
    load(1, 1)
    sissue(1)

    def body(k, _):
        @pl.when(k % 2 == 0)
        def _e():
            swait(0)
            load(k, 0)
            sissue(0)

        @pl.when(k % 2 == 1)
        def _o():
            swait(1)
            load(k, 1)
            sissue(1)
        return 0
    lax.fori_loop(2, nck, body, 0)
    swait((nck - 2) % 2)
    swait((nck - 1) % 2)

    plsc.subcore_barrier()

    @pl.when(c == 0)
    def _out0():
        _stage_out(zbuf, CHD, acc, dega, s)

    @pl.when(c == 1)
    def _out1():
        _stage_out(zbuf, CHD, acc, degb, s)


def _aggregate(src_hbm, dst_hbm, y_hbm, acc, bufs, sidx, didx, gsem, ssem,
               base, nck, ch):
    def load(k, p):
        pltpu.sync_copy(src_hbm.at[pl.ds(base + k * ch, ch)], sidx[p])
        pltpu.sync_copy(dst_hbm.at[pl.ds(base + k * ch, ch)], didx[p])

    def gissue(p):
        pltpu.async_copy(y_hbm.at[sidx[p]], bufs[p], gsem[p])

    def gwait(p):
        pltpu.make_async_copy(y_hbm.at[sidx[p]], bufs[p], gsem[p]).wait()

    def sissue(p):
        pltpu.async_copy(bufs[p], acc.at[didx[p]], ssem[p], add=True)

    def swait(p):
        pltpu.make_async_copy(bufs[p], acc.at[didx[p]], ssem[p]).wait()

    load(0, 0)
    gissue(0)
    load(1, 1)
    gissue(1)
    gwait(0)
    sissue(0)

    def body(k, _):
        @pl.when(k % 2 == 0)
        def _e():
            swait(0)
            load(k, 0)
            gissue(0)
            gwait(1)
            sissue(1)

        @pl.when(k % 2 == 1)
        def _o():
            swait(1)
            load(k, 1)
            gissue(1)
            gwait(0)
            sissue(0)
        return 0
    lax.fori_loop(2, nck, body, 0)

    p1 = (nck - 1) % 2
    swait(1 - p1)
    gwait(p1)
    sissue(p1)
    swait(p1)


def _agg_scratch(ch):
    return [
        pltpu.VMEM((ch, 16), f32),
        pltpu.VMEM((ch, 16), f32),
        pltpu.VMEM((ch,), i32),
        pltpu.VMEM((ch,), i32),
        pltpu.VMEM((ch,), i32),
        pltpu.VMEM((ch,), i32),
        pltpu.VMEM_SHARED((N, 16), f32),
        pltpu.SemaphoreType.DMA,
        pltpu.SemaphoreType.DMA,
        pltpu.SemaphoreType.DMA,
        pltpu.SemaphoreType.DMA,
    ]


CH1 = 800


@functools.partial(
    pl.kernel,
    out_type=[jax.ShapeDtypeStruct((N, 16), f32),
              jax.ShapeDtypeStruct((N, 16), f32)],
    mesh=_MESH,
    compiler_params=_SC_PARAMS,
    scratch_types=_agg_scratch(CH1),
)
def _agg_feature_split(src_hbm, dst_hbm, ya_hbm, yb_hbm, agga, aggb,
                       bufa, bufb, sxa, sxb, dxa, dxb, acc, g0, g1, s0, s1):
    c = lax.axis_index("c")
    s = lax.axis_index("s")
    _fill(bufa, CH1, 0.0)
    _stage_zero(bufa, CH1, acc, s)
    plsc.subcore_barrier()

    base = s * (E // 16)
    nck = (E // 16) // CH1

    @pl.when(c == 0)
    def _run0():
        _aggregate(src_hbm, dst_hbm, ya_hbm, acc, (bufa, bufb),
                   (sxa, sxb), (dxa, dxb), (g0, g1), (s0, s1),
                   base, nck, CH1)

    @pl.when(c == 1)
    def _run1():
        _aggregate(src_hbm, dst_hbm, yb_hbm, acc, (bufa, bufb),
                   (sxa, sxb), (dxa, dxb), (g0, g1), (s0, s1),
                   base, nck, CH1)

    plsc.subcore_barrier()

    @pl.when(c == 0)
    def _out0():
        _stage_out(bufa, CH1, acc, agga, s)

    @pl.when(c == 1)
    def _out1():
        _stage_out(bufa, CH1, acc, aggb, s)


CH2 = 400


@functools.partial(
    pl.kernel,
    out_type=[jax.ShapeDtypeStruct((N, 16), f32),
              jax.ShapeDtypeStruct((N, 16), f32)],
    mesh=_MESH,
    compiler_params=_SC_PARAMS,
    scratch_types=_agg_scratch(CH2),
)
def _agg_edge_split(src_hbm, dst_hbm, y_hbm, agga, aggb,
                    bufa, bufb, sxa, sxb, dxa, dxb, acc, g0, g1, s0, s1):
    c = lax.axis_index("c")
    s = lax.axis_index("s")
    _fill(bufa, CH2, 0.0)
    _stage_zero(bufa, CH2, acc, s)
    plsc.subcore_barrier()

    base = c * (E // 2) + s * (E // 32)
    nck = (E // 32) // CH2
    _aggregate(src_hbm, dst_hbm, y_hbm, acc, (bufa, bufb),
               (sxa, sxb), (dxa, dxb), (g0, g1), (s0, s1), base, nck, CH2)

    plsc.subcore_barrier()

    @pl.when(c == 0)
    def _out0():
        _stage_out(bufa, CH2, acc, agga, s)

    @pl.when(c == 1)
    def _out1():
        _stage_out(bufa, CH2, acc, aggb, s)


R = 2000


def _tc_a_body(dega, degb, x, w1, dinv16, y1a, y1b):
    deg = dega[:, :1] + degb[:, :1] + 1.0
    dinv = lax.rsqrt(deg)
    xw = jnp.dot(x[...], w1[...], preferred_element_type=f32)
    y = xw * dinv
    dinv16[...] = jnp.broadcast_to(dinv, (R, 16))
    y1a[...] = y[:, :16]
    y1b[...] = y[:, 16:]


def _tc_b_body(dinv16, agg1a, agg1b, y1a, y1b, w2, b1, y2):
    d = dinv16[...]
    h = jnp.concatenate([d * (agg1a[...] + y1a[...]),
                         d * (agg1b[...] + y1b[...])], axis=1)
    h = jnp.maximum(h + b1[...][None, :], 0.0)
    y2[...] = jnp.dot(h, w2[...], preferred_element_type=f32) * d


def _tc_c_body(dinv16, agg2a, agg2b, y2, b2, out):
    res = dinv16[...] * (agg2a[...] + agg2b[...] + y2[...]) + b2[...][None, :]
    out[...] = res[:, :10]


def _rb(cols):
    return pl.BlockSpec((R, cols), lambda i: (i, 0))


def _full(shape):
    nd = len(shape)
    return pl.BlockSpec(shape, lambda i: (0,) * nd)


def kernel(features, edge_index, W1, b1, W2, b2):
    src = edge_index[0].astype(i32)
    dst = edge_index[1].astype(i32)
    w2p = jnp.pad(W2, ((0, 0), (0, C_PAD - 10)))
    b2p = jnp.pad(b2, (0, C_PAD - 10))

    dega, degb = _deg_kernel(dst)

    dinv16, y1a, y1b = pl.pallas_call(
        _tc_a_body,
        grid=(N // R,),
        in_specs=[_rb(16), _rb(16), _rb(F_IN), _full((F_IN, 32))],
        out_specs=[_rb(16), _rb(16), _rb(16)],
        out_shape=[jax.ShapeDtypeStruct((N, 16), f32)] * 3,
    )(dega, degb, features, W1)

    agg1a, agg1b = _agg_feature_split(src, dst, y1a, y1b)

    y2 = pl.pallas_call(
        _tc_b_body,
        grid=(N // R,),
        in_specs=[_rb(16)] * 5 + [_full((32, C_PAD)), _full((32,))],
        out_specs=_rb(C_PAD),
        out_shape=jax.ShapeDtypeStruct((N, C_PAD), f32),
    )(dinv16, agg1a, agg1b, y1a, y1b, w2p, b1)

    agg2a, agg2b = _agg_edge_split(src, dst, y2)

    out = pl.pallas_call(
        _tc_c_body,
        grid=(N // R,),
        in_specs=[_rb(16)] * 4 + [_full((C_PAD,))],
        out_specs=_rb(10),
        out_shape=jax.ShapeDtypeStruct((N, 10), f32),
    )(dinv16, agg2a, agg2b, y2, b2p)

    return out


# speedup vs baseline: 44.1113x; 1.5798x over previous
import functools

import jax
import jax.numpy as jnp
from jax import lax
from jax.experimental import pallas as pl
from jax.experimental.pallas import tpu as pltpu
from jax.experimental.pallas import tpu_sc as plsc

N = 100000
E = 1600000
N2 = 102400
NP = N2 // 8
NT = N // 16

f32 = jnp.float32
i32 = jnp.int32

_MESH = plsc.VectorSubcoreMesh(core_axis_name="c", subcore_axis_name="s")
_SC_PARAMS = pltpu.CompilerParams(use_tc_tiling_on_sc=False)


def _fill(ref, rows, value):
    def body(i, _):
        ref[i, :] = jnp.full((16,), value, f32)
        return 0
    lax.fori_loop(0, rows, body, 0)


def _chunks(rows_buf):
    full, rem = NT // rows_buf, NT % rows_buf
    out = [(m * rows_buf, rows_buf) for m in range(full)]
    if rem:
        out.append((full * rows_buf, rem))
    return out


def _stage_zero(zbuf, rows_buf, acc, s, sem):
    descs = []
    for off, sz in _chunks(rows_buf):
        descs.append(pltpu.async_copy(
            zbuf.at[pl.ds(0, sz)], acc.at[pl.ds(s * NT + off, sz)], sem))
    for d in descs:
        d.wait()


def _stage_out(bufs, rows_buf, acc, out_ref, s, isems, osems):
    ck = _chunks(rows_buf)
    ind = [None, None]
    outd = [None, None]
    for idx, (off, sz) in enumerate(ck):
        p = idx & 1
        if outd[p] is not None:
            outd[p].wait()
        sl = pl.ds(s * NT + off, sz)
        ind[p] = pltpu.async_copy(acc.at[sl], bufs[p].at[pl.ds(0, sz)],
                                  isems[p])
        if idx > 0:
            q = (idx - 1) & 1
            ind[q].wait()
            poff, psz = ck[idx - 1]
            outd[q] = pltpu.async_copy(
                bufs[q].at[pl.ds(0, psz)],
                out_ref.at[pl.ds(s * NT + poff, psz)], osems[q])
    lastp = (len(ck) - 1) & 1
    ind[lastp].wait()
    loff, lsz = ck[-1]
    outd[lastp] = pltpu.async_copy(
        bufs[lastp].at[pl.ds(0, lsz)],
        out_ref.at[pl.ds(s * NT + loff, lsz)], osems[lastp])
    if len(ck) > 1 and outd[1 - lastp] is not None:
        outd[1 - lastp].wait()
    outd[lastp].wait()


CHD = 400


@functools.partial(
    pl.kernel,
    out_type=[jax.ShapeDtypeStruct((N2, 16), f32),
              jax.ShapeDtypeStruct((N2, 16), f32)],
    mesh=_MESH,
    compiler_params=_SC_PARAMS,
    scratch_types=[
        pltpu.VMEM((CHD, 16), f32),
        pltpu.VMEM((CHD, 16), f32),
        pltpu.VMEM((CHD,), i32),
        pltpu.VMEM((CHD,), i32),
        pltpu.VMEM_SHARED((N, 16), f32),
        pltpu.SemaphoreType.DMA,
        pltpu.SemaphoreType.DMA,
        pltpu.SemaphoreType.DMA,
        pltpu.SemaphoreType.DMA,
    ],
)
def _deg_kernel(dst_hbm, dega, degb, zbuf, ones, dxa, dxb, acc,
                sma, smb, smc, smd):
    c = lax.axis_index("c")
    s = lax.axis_index("s")
    _fill(zbuf, CHD, 0.0)
    _fill(ones, CHD, 1.0)
    _stage_zero(zbuf, CHD, acc, s, sma)
    plsc.subcore_barrier()

    base = c * (E // 2) + s * (E // 32)
    nck = (E // 32) // CHD
    didx = (dxa, dxb)
    sems = (sma, smb)

    def load(k, p):
        pltpu.sync_copy(dst_hbm.at[pl.ds(base + k * CHD, CHD)], didx[p])

    def sissue(p):
        pltpu.async_copy(ones, acc.at[didx[p]], sems[p], add=True)

    def swait(p):
        pltpu.make_async_copy(ones, acc.at[didx[p]], sems[p]).wait()

    load(0, 0)
    sissue(0)
    load(1, 1)
    sissue(1)

    def body(k, _):
        @pl.when(k % 2 == 0)
        def _e():
            swait(0)
            load(k, 0)
            sissue(0)

        @pl.when(k % 2 == 1)
        def _o():
            swait(1)
            load(k, 1)
            sissue(1)
        return 0
    lax.fori_loop(2, nck, body, 0)
    swait((nck - 2) % 2)
    swait((nck - 1) % 2)

    plsc.subcore_barrier()

    @pl.when(c == 0)
    def _out0():
        _stage_out((zbuf, ones), CHD, acc, dega, s, (sma, smb), (smc, smd))

    @pl.when(c == 1)
    def _out1():
        _stage_out((zbuf, ones), CHD, acc, degb, s, (sma, smb), (smc, smd))


def _aggregate(src_hbm, dst_hbm, y_hbm, acc, bufs, sidx, didx, gsem, ssem,
               base, nck, ch):
    def load(k, p):
        pltpu.sync_copy(src_hbm.at[pl.ds(base + k * ch, ch)], sidx[p])
        pltpu.sync_copy(dst_hbm.at[pl.ds(base + k * ch, ch)], didx[p])

    def gissue(p):
        pltpu.async_copy(y_hbm.at[sidx[p]], bufs[p], gsem[p])

    def gwait(p):
        pltpu.make_async_copy(y_hbm.at[sidx[p]], bufs[p], gsem[p]).wait()

    def sissue(p):
        pltpu.async_copy(bufs[p], acc.at[didx[p]], ssem[p], add=True)

    def swait(p):
        pltpu.make_async_copy(bufs[p], acc.at[didx[p]], ssem[p]).wait()

    load(0, 0)
    gissue(0)
    load(1, 1)
    gissue(1)
    gwait(0)
    sissue(0)

    def body(k, _):
        @pl.when(k % 2 == 0)
        def _e():
            swait(0)
            load(k, 0)
            gissue(0)
            gwait(1)
            sissue(1)

        @pl.when(k % 2 == 1)
        def _o():
            swait(1)
            load(k, 1)
            gissue(1)
            gwait(0)
            sissue(0)
        return 0
    lax.fori_loop(2, nck, body, 0)

    p1 = (nck - 1) % 2
    swait(1 - p1)
    gwait(p1)
    sissue(p1)
    swait(p1)


def _agg_scratch(ch):
    return [
        pltpu.VMEM((ch, 16), f32),
        pltpu.VMEM((ch, 16), f32),
        pltpu.VMEM((ch,), i32),
        pltpu.VMEM((ch,), i32),
        pltpu.VMEM((ch,), i32),
        pltpu.VMEM((ch,), i32),
        pltpu.VMEM_SHARED((N, 16), f32),
        pltpu.SemaphoreType.DMA,
        pltpu.SemaphoreType.DMA,
        pltpu.SemaphoreType.DMA,
        pltpu.SemaphoreType.DMA,
    ]


CH1 = 800


@functools.partial(
    pl.kernel,
    out_type=[jax.ShapeDtypeStruct((N2, 16), f32),
              jax.ShapeDtypeStruct((N2, 16), f32)],
    mesh=_MESH,
    compiler_params=_SC_PARAMS,
    scratch_types=_agg_scratch(CH1),
)
def _agg_feature_split(src_hbm, dst_hbm, ya_hbm, yb_hbm, agga, aggb,
                       bufa, bufb, sxa, sxb, dxa, dxb, acc, g0, g1, s0, s1):
    c = lax.axis_index("c")
    s = lax.axis_index("s")
    _fill(bufa, CH1, 0.0)
    _stage_zero(bufa, CH1, acc, s, g0)
    plsc.subcore_barrier()

    base = s * (E // 16)
    nck = (E // 16) // CH1

    @pl.when(c == 0)
    def _run0():
        _aggregate(src_hbm, dst_hbm, ya_hbm, acc, (bufa, bufb),
                   (sxa, sxb), (dxa, dxb), (g0, g1), (s0, s1),
                   base, nck, CH1)

    @pl.when(c == 1)
    def _run1():
        _aggregate(src_hbm, dst_hbm, yb_hbm, acc, (bufa, bufb),
                   (sxa, sxb), (dxa, dxb), (g0, g1), (s0, s1),
                   base, nck, CH1)

    plsc.subcore_barrier()

    @pl.when(c == 0)
    def _out0():
        _stage_out((bufa, bufb), CH1, acc, agga, s, (g0, g1), (s0, s1))

    @pl.when(c == 1)
    def _out1():
        _stage_out((bufa, bufb), CH1, acc, aggb, s, (g0, g1), (s0, s1))


CH2 = 400


@functools.partial(
    pl.kernel,
    out_type=[jax.ShapeDtypeStruct((N2, 16), f32),
              jax.ShapeDtypeStruct((N2, 16), f32)],
    mesh=_MESH,
    compiler_params=_SC_PARAMS,
    scratch_types=_agg_scratch(CH2),
)
def _agg_edge_split(src_hbm, dst_hbm, y_hbm, agga, aggb,
                    bufa, bufb, sxa, sxb, dxa, dxb, acc, g0, g1, s0, s1):
    c = lax.axis_index("c")
    s = lax.axis_index("s")
    _fill(bufa, CH2, 0.0)
    _stage_zero(bufa, CH2, acc, s, g0)
    plsc.subcore_barrier()

    base = c * (E // 2) + s * (E // 32)
    nck = (E // 32) // CH2
    _aggregate(src_hbm, dst_hbm, y_hbm, acc, (bufa, bufb),
               (sxa, sxb), (dxa, dxb), (g0, g1), (s0, s1), base, nck, CH2)

    plsc.subcore_barrier()

    @pl.when(c == 0)
    def _out0():
        _stage_out((bufa, bufb), CH2, acc, agga, s, (g0, g1), (s0, s1))

    @pl.when(c == 1)
    def _out1():
        _stage_out((bufa, bufb), CH2, acc, aggb, s, (g0, g1), (s0, s1))


PB = 1600


def _tc_a_body(dega, degb, xp, w1a, w1b, dinv16, y1a, y1b):
    deg = dega[...] + degb[...] + 1.0
    dinv = lax.rsqrt(deg)
    dinv16[...] = dinv
    x = xp[...]
    y1a[...] = jnp.dot(x, w1a[...], preferred_element_type=f32) * dinv
    y1b[...] = jnp.dot(x, w1b[...], preferred_element_type=f32) * dinv


def _tc_b_body(dinv16, agg1a, agg1b, y1a, y1b, w2a, w2b, b1a, b1b, y2):
    d = dinv16[...]
    ha = jnp.maximum(d * (agg1a[...] + y1a[...]) + b1a[...][None, :], 0.0)
    hb = jnp.maximum(d * (agg1b[...] + y1b[...]) + b1b[...][None, :], 0.0)
    y2[...] = (jnp.dot(ha, w2a[...], preferred_element_type=f32)
               + jnp.dot(hb, w2b[...], preferred_element_type=f32)) * d


def _tc_c_body(dinv16, agg2a, agg2b, y2, b2t, out):
    out[...] = (dinv16[...] * (agg2a[...] + agg2b[...] + y2[...])
                + b2t[...][None, :])


def _pb():
    return pl.BlockSpec((PB, 128), lambda i: (i, 0))


def _full(shape):
    nd = len(shape)
    return pl.BlockSpec(shape, lambda i: (0,) * nd)


def _packed(a):
    return a.reshape(NP, 128)


def kernel(features, edge_index, W1, b1, W2, b2):
    src = edge_index[0].astype(i32)
    dst = edge_index[1].astype(i32)

    xp = _packed(jnp.pad(features, ((0, N2 - N), (0, 6))))
    eye8 = jnp.eye(8, dtype=f32)
    w1p = jnp.pad(W1, ((0, 6), (0, 0)))
    w1a = jnp.kron(eye8, w1p[:, :16])
    w1b = jnp.kron(eye8, w1p[:, 16:])
    w2p = jnp.pad(W2, ((0, 0), (0, 6)))
    w2a = jnp.kron(eye8, w2p[:16, :])
    w2b = jnp.kron(eye8, w2p[16:, :])
    b1a = jnp.tile(b1[:16], 8)
    b1b = jnp.tile(b1[16:], 8)
    b2t = jnp.tile(jnp.pad(b2, (0, 6)), 8)

    dega, degb = _deg_kernel(dst)

    dinv16, y1a, y1b = pl.pallas_call(
        _tc_a_body,
        grid=(NP // PB,),
        in_specs=[_pb(), _pb(), _pb(), _full((128, 128)), _full((128, 128))],
        out_specs=[_pb(), _pb(), _pb()],
        out_shape=[jax.ShapeDtypeStruct((NP, 128), f32)] * 3,
    )(_packed(dega), _packed(degb), xp, w1a, w1b)

    agg1a, agg1b = _agg_feature_split(src, dst,
                                      y1a.reshape(N2, 16), y1b.reshape(N2, 16))

    y2 = pl.pallas_call(
        _tc_b_body,
        grid=(NP // PB,),
        in_specs=[_pb()] * 5 + [_full((128, 128))] * 2 + [_full((128,))] * 2,
        out_specs=_pb(),
        out_shape=jax.ShapeDtypeStruct((NP, 128), f32),
    )(dinv16, _packed(agg1a), _packed(agg1b), y1a, y1b, w2a, w2b, b1a, b1b)

    agg2a, agg2b = _agg_edge_split(src, dst, y2.reshape(N2, 16))

    outp = pl.pallas_call(
        _tc_c_body,
        grid=(NP // PB,),
        in_specs=[_pb()] * 4 + [_full((128,))],
        out_specs=_pb(),
        out_shape=jax.ShapeDtypeStruct((NP, 128), f32),
    )(dinv16, _packed(agg2a), _packed(agg2b), y2, b2t)

    return outp.reshape(N2, 16)[:N, :10]


# speedup vs baseline: 51.8622x; 1.1757x over previous
import functools

import jax
import jax.numpy as jnp
from jax import lax
from jax.experimental import pallas as pl
from jax.experimental.pallas import tpu as pltpu
from jax.experimental.pallas import tpu_sc as plsc

N = 100000
E = 1600000
N2 = 102400
NP = N2 // 8
NT = N // 16

f32 = jnp.float32
i32 = jnp.int32

_MESH = plsc.VectorSubcoreMesh(core_axis_name="c", subcore_axis_name="s")
_SC_PARAMS = pltpu.CompilerParams(use_tc_tiling_on_sc=False)


def _fill(ref, rows, value):
    def body(i, _):
        ref[i, :] = jnp.full((16,), value, f32)
        return 0
    lax.fori_loop(0, rows, body, 0)


def _stage_zero(zbuf, rows_buf, acc, s, sem):
    full, rem = NT // rows_buf, NT % rows_buf
    descs = []
    for m in range(full):
        descs.append(pltpu.async_copy(
            zbuf, acc.at[pl.ds(s * NT + m * rows_buf, rows_buf)], sem))
    if rem:
        descs.append(pltpu.async_copy(
            zbuf.at[pl.ds(0, rem)],
            acc.at[pl.ds(s * NT + full * rows_buf, rem)], sem))
    for d in descs:
        d.wait()


def _acc_out(acc, out_ref, s):
    sl = pl.ds(s * NT, NT)
    pltpu.sync_copy(acc.at[sl], out_ref.at[sl])


CHD = 800


@functools.partial(
    pl.kernel,
    out_type=[jax.ShapeDtypeStruct((N2, 16), f32),
              jax.ShapeDtypeStruct((N2, 16), f32)],
    mesh=_MESH,
    compiler_params=_SC_PARAMS,
    scratch_types=[
        pltpu.VMEM((CHD, 16), f32),
        pltpu.VMEM((CHD, 16), f32),
        pltpu.VMEM((CHD,), i32),
        pltpu.VMEM((CHD,), i32),
        pltpu.VMEM((400,), i32),
        pltpu.VMEM_SHARED((N, 16), f32),
        pltpu.SemaphoreType.DMA,
        pltpu.SemaphoreType.DMA,
    ],
)
def _deg_kernel(edge_hbm, dega, degb, zbuf, ones, dxa, dxb, dxt, acc,
                sma, smb):
    c = lax.axis_index("c")
    s = lax.axis_index("s")
    _fill(zbuf, CHD, 0.0)
    _fill(ones, CHD, 1.0)
    _stage_zero(zbuf, CHD, acc, s, sma)
    plsc.subcore_barrier()

    base = c * (E // 2) + s * (E // 32)
    nck = 62
    didx = (dxa, dxb)
    sems = (sma, smb)

    def load(k, p):
        pltpu.sync_copy(edge_hbm.at[1, pl.ds(base + k * CHD, CHD)], didx[p])

    def sissue(p):
        pltpu.async_copy(ones, acc.at[didx[p]], sems[p], add=True)

    def swait(p):
        pltpu.make_async_copy(ones, acc.at[didx[p]], sems[p]).wait()

    load(0, 0)
    sissue(0)
    load(1, 1)
    sissue(1)

    def body(k, _):
        @pl.when(k % 2 == 0)
        def _e():
            swait(0)
            load(k, 0)
            sissue(0)

        @pl.when(k % 2 == 1)
        def _o():
            swait(1)
            load(k, 1)
            sissue(1)
        return 0
    lax.fori_loop(2, nck, body, 0)
    swait(nck % 2)
    swait(1 - nck % 2)

    pltpu.sync_copy(edge_hbm.at[1, pl.ds(base + nck * CHD, 400)], dxt)
    pltpu.sync_copy(ones.at[pl.ds(0, 400)], acc.at[dxt], add=True)

    plsc.subcore_barrier()

    @pl.when(c == 0)
    def _out0():
        _acc_out(acc, dega, s)

    @pl.when(c == 1)
    def _out1():
        _acc_out(acc, degb, s)


def _aggregate(edge_hbm, y_hbm, acc, bufs, sidx, didx, gsem, ssem,
               base, nck, ch, tail, tbufs):
    def load(k, p):
        pltpu.sync_copy(edge_hbm.at[0, pl.ds(base + k * ch, ch)], sidx[p])
        pltpu.sync_copy(edge_hbm.at[1, pl.ds(base + k * ch, ch)], didx[p])

    def gissue(p):
        pltpu.async_copy(y_hbm.at[sidx[p]], bufs[p], gsem[p])

    def gwait(p):
        pltpu.make_async_copy(y_hbm.at[sidx[p]], bufs[p], gsem[p]).wait()

    def sissue(p):
        pltpu.async_copy(bufs[p], acc.at[didx[p]], ssem[p], add=True)

    def swait(p):
        pltpu.make_async_copy(bufs[p], acc.at[didx[p]], ssem[p]).wait()

    load(0, 0)
    gissue(0)
    load(1, 1)
    gissue(1)
    gwait(0)
    sissue(0)

    def body(k, _):
        @pl.when(k % 2 == 0)
        def _e():
            swait(0)
            load(k, 0)
            gissue(0)
            gwait(1)
            sissue(1)

        @pl.when(k % 2 == 1)
        def _o():
            swait(1)
            load(k, 1)
            gissue(1)
            gwait(0)
            sissue(0)
        return 0
    lax.fori_loop(2, nck, body, 0)

    p1 = (nck - 1) % 2
    swait(1 - p1)
    gwait(p1)
    sissue(p1)
    swait(p1)

    if tail:
        sxt, dxt = tbufs
        tb = bufs[0].at[pl.ds(0, tail)]
        off = base + nck * ch
        pltpu.sync_copy(edge_hbm.at[0, pl.ds(off, tail)], sxt)
        pltpu.sync_copy(edge_hbm.at[1, pl.ds(off, tail)], dxt)
        pltpu.async_copy(y_hbm.at[sxt], tb, gsem[0]).wait()
        pltpu.sync_copy(tb, acc.at[dxt], add=True)


def _agg_scratch(ch, tail):
    sc = [
        pltpu.VMEM((ch, 16), f32),
        pltpu.VMEM((ch, 16), f32),
        pltpu.VMEM((ch,), i32),
        pltpu.VMEM((ch,), i32),
        pltpu.VMEM((ch,), i32),
        pltpu.VMEM((ch,), i32),
        pltpu.VMEM_SHARED((N, 16), f32),
        pltpu.SemaphoreType.DMA,
        pltpu.SemaphoreType.DMA,
        pltpu.SemaphoreType.DMA,
        pltpu.SemaphoreType.DMA,
    ]
    if tail:
        sc.insert(6, pltpu.VMEM((tail,), i32))
        sc.insert(7, pltpu.VMEM((tail,), i32))
    return sc


CH1 = 800


@functools.partial(
    pl.kernel,
    out_type=[jax.ShapeDtypeStruct((N2, 16), f32),
              jax.ShapeDtypeStruct((N2, 16), f32)],
    mesh=_MESH,
    compiler_params=_SC_PARAMS,
    scratch_types=_agg_scratch(CH1, 0),
)
def _agg_feature_split(edge_hbm, ya_hbm, yb_hbm, agga, aggb,
                       bufa, bufb, sxa, sxb, dxa, dxb, acc, g0, g1, s0, s1):
    c = lax.axis_index("c")
    s = lax.axis_index("s")
    _fill(bufa, CH1, 0.0)
    _stage_zero(bufa, CH1, acc, s, g0)
    plsc.subcore_barrier()

    base = s * (E // 16)
    nck = (E // 16) // CH1

    @pl.when(c == 0)
    def _run0():
        _aggregate(edge_hbm, ya_hbm, acc, (bufa, bufb),
                   (sxa, sxb), (dxa, dxb), (g0, g1), (s0, s1),
                   base, nck, CH1, 0, None)

    @pl.when(c == 1)
    def _run1():
        _aggregate(edge_hbm, yb_hbm, acc, (bufa, bufb),
                   (sxa, sxb), (dxa, dxb), (g0, g1), (s0, s1),
                   base, nck, CH1, 0, None)

    plsc.subcore_barrier()

    @pl.when(c == 0)
    def _out0():
        _acc_out(acc, agga, s)

    @pl.when(c == 1)
    def _out1():
        _acc_out(acc, aggb, s)


CH2 = 800


@functools.partial(
    pl.kernel,
    out_type=[jax.ShapeDtypeStruct((N2, 16), f32),
              jax.ShapeDtypeStruct((N2, 16), f32)],
    mesh=_MESH,
    compiler_params=_SC_PARAMS,
    scratch_types=_agg_scratch(CH2, 400),
)
def _agg_edge_split(edge_hbm, y_hbm, agga, aggb,
                    bufa, bufb, sxa, sxb, dxa, dxb, sxt, dxt, acc,
                    g0, g1, s0, s1):
    c = lax.axis_index("c")
    s = lax.axis_index("s")
    _fill(bufa, CH2, 0.0)
    _stage_zero(bufa, CH2, acc, s, g0)
    plsc.subcore_barrier()

    base = c * (E // 2) + s * (E // 32)
    _aggregate(edge_hbm, y_hbm, acc, (bufa, bufb),
               (sxa, sxb), (dxa, dxb), (g0, g1), (s0, s1),
               base, 62, CH2, 400, (sxt, dxt))

    plsc.subcore_barrier()

    @pl.when(c == 0)
    def _out0():
        _acc_out(acc, agga, s)

    @pl.when(c == 1)
    def _out1():
        _acc_out(acc, aggb, s)


PB = 1600


def _tc_a_body(dega, degb, xp, w1a, w1b, dinv16, y1a, y1b):
    deg = dega[...] + degb[...] + 1.0
    dinv = lax.rsqrt(deg)
    dinv16[...] = dinv
    x = xp[...]
    y1a[...] = jnp.dot(x, w1a[...], preferred_element_type=f32) * dinv
    y1b[...] = jnp.dot(x, w1b[...], preferred_element_type=f32) * dinv


def _tc_b_body(dinv16, agg1a, agg1b, y1a, y1b, w2a, w2b, b1a, b1b, y2):
    d = dinv16[...]
    ha = jnp.maximum(d * (agg1a[...] + y1a[...]) + b1a[...][None, :], 0.0)
    hb = jnp.maximum(d * (agg1b[...] + y1b[...]) + b1b[...][None, :], 0.0)
    y2[...] = (jnp.dot(ha, w2a[...], preferred_element_type=f32)
               + jnp.dot(hb, w2b[...], preferred_element_type=f32)) * d


def _tc_c_body(dinv16, agg2a, agg2b, y2, b2t, out):
    out[...] = (dinv16[...] * (agg2a[...] + agg2b[...] + y2[...])
                + b2t[...][None, :])


def _pb():
    return pl.BlockSpec((PB, 128), lambda i: (i, 0))


def _full(shape):
    nd = len(shape)
    return pl.BlockSpec(shape, lambda i: (0,) * nd)


def _packed(a):
    return a.reshape(NP, 128)


def kernel(features, edge_index, W1, b1, W2, b2):
    edges = edge_index.astype(i32)

    xp = _packed(jnp.pad(features, ((0, N2 - N), (0, 6))))
    eye8 = jnp.eye(8, dtype=f32)
    w1p = jnp.pad(W1, ((0, 6), (0, 0)))
    w1a = jnp.kron(eye8, w1p[:, :16])
    w1b = jnp.kron(eye8, w1p[:, 16:])
    w2p = jnp.pad(W2, ((0, 0), (0, 6)))
    w2a = jnp.kron(eye8, w2p[:16, :])
    w2b = jnp.kron(eye8, w2p[16:, :])
    b1a = jnp.tile(b1[:16], 8)
    b1b = jnp.tile(b1[16:], 8)
    b2t = jnp.tile(jnp.pad(b2, (0, 6)), 8)

    dega, degb = _deg_kernel(edges)

    dinv16, y1a, y1b = pl.pallas_call(
        _tc_a_body,
        grid=(NP // PB,),
        in_specs=[_pb(), _pb(), _pb(), _full((128, 128)), _full((128, 128))],
        out_specs=[_pb(), _pb(), _pb()],
        out_shape=[jax.ShapeDtypeStruct((NP, 128), f32)] * 3,
    )(_packed(dega), _packed(degb), xp, w1a, w1b)

    agg1a, agg1b = _agg_feature_split(edges, y1a.reshape(N2, 16),
                                      y1b.reshape(N2, 16))

    y2 = pl.pallas_call(
        _tc_b_body,
        grid=(NP // PB,),
        in_specs=[_pb()] * 5 + [_full((128, 128))] * 2 + [_full((128,))] * 2,
        out_specs=_pb(),
        out_shape=jax.ShapeDtypeStruct((NP, 128), f32),
    )(dinv16, _packed(agg1a), _packed(agg1b), y1a, y1b, w2a, w2b, b1a, b1b)

    agg2a, agg2b = _agg_edge_split(edges, y2.reshape(N2, 16))

    outp = pl.pallas_call(
        _tc_c_body,
        grid=(NP // PB,),
        in_specs=[_pb()] * 4 + [_full((128,))],
        out_specs=_pb(),
        out_shape=jax.ShapeDtypeStruct((NP, 128), f32),
    )(dinv16, _packed(agg2a), _packed(agg2b), y2, b2t)

    return outp.reshape(N2, 16)[:N, :10]


# speedup vs baseline: 51.8716x; 1.0002x over previous
import functools

import jax
import jax.numpy as jnp
from jax import lax
from jax.experimental import pallas as pl
from jax.experimental.pallas import tpu as pltpu
from jax.experimental.pallas import tpu_sc as plsc

N = 100000
E = 1600000
N2 = 102400
NP = N2 // 8
NT = N // 16

f32 = jnp.float32
i32 = jnp.int32

_MESH = plsc.VectorSubcoreMesh(core_axis_name="c", subcore_axis_name="s")
_SC_PARAMS = pltpu.CompilerParams(use_tc_tiling_on_sc=False)
_TC_PARAMS = pltpu.CompilerParams(needs_layout_passes=False)


def _fill(ref, rows, value):
    def body(i, _):
        ref[i, :] = jnp.full((16,), value, f32)
        return 0
    lax.fori_loop(0, rows, body, 0)


def _stage_zero(zbuf, rows_buf, acc, s, sem):
    full, rem = NT // rows_buf, NT % rows_buf
    descs = []
    for m in range(full):
        descs.append(pltpu.async_copy(
            zbuf, acc.at[pl.ds(s * NT + m * rows_buf, rows_buf)], sem))
    if rem:
        descs.append(pltpu.async_copy(
            zbuf.at[pl.ds(0, rem)],
            acc.at[pl.ds(s * NT + full * rows_buf, rem)], sem))
    for d in descs:
        d.wait()


def _acc_out(acc, out_ref, s):
    sl = pl.ds(s * NT, NT)
    pltpu.sync_copy(acc.at[sl], out_ref.at[sl])


CHD = 800


@functools.partial(
    pl.kernel,
    out_type=[jax.ShapeDtypeStruct((N2, 16), f32),
              jax.ShapeDtypeStruct((N2, 16), f32)],
    mesh=_MESH,
    compiler_params=_SC_PARAMS,
    scratch_types=[
        pltpu.VMEM((CHD, 16), f32),
        pltpu.VMEM((CHD, 16), f32),
        pltpu.VMEM((CHD,), i32),
        pltpu.VMEM((CHD,), i32),
        pltpu.VMEM((400,), i32),
        pltpu.VMEM_SHARED((N, 16), f32),
        pltpu.SemaphoreType.DMA,
        pltpu.SemaphoreType.DMA,
    ],
)
def _deg_kernel(edge_hbm, dega, degb, zbuf, ones, dxa, dxb, dxt, acc,
                sma, smb):
    c = lax.axis_index("c")
    s = lax.axis_index("s")
    _fill(zbuf, CHD, 0.0)
    _fill(ones, CHD, 1.0)
    _stage_zero(zbuf, CHD, acc, s, sma)
    plsc.subcore_barrier()

    base = c * (E // 2) + s * (E // 32)
    nck = 62
    didx = (dxa, dxb)
    sems = (sma, smb)

    def load(k, p):
        pltpu.sync_copy(edge_hbm.at[1, pl.ds(base + k * CHD, CHD)], didx[p])

    def sissue(p):
        pltpu.async_copy(ones, acc.at[didx[p]], sems[p], add=True)

    def swait(p):
        pltpu.make_async_copy(ones, acc.at[didx[p]], sems[p]).wait()

    load(0, 0)
    sissue(0)
    load(1, 1)
    sissue(1)

    def body(k, _):
        @pl.when(k % 2 == 0)
        def _e():
            swait(0)
            load(k, 0)
            sissue(0)

        @pl.when(k % 2 == 1)
        def _o():
            swait(1)
            load(k, 1)
            sissue(1)
        return 0
    lax.fori_loop(2, nck, body, 0)
    swait(nck % 2)
    swait(1 - nck % 2)

    pltpu.sync_copy(edge_hbm.at[1, pl.ds(base + nck * CHD, 400)], dxt)
    pltpu.sync_copy(ones.at[pl.ds(0, 400)], acc.at[dxt], add=True)

    plsc.subcore_barrier()

    @pl.when(c == 0)
    def _out0():
        _acc_out(acc, dega, s)

    @pl.when(c == 1)
    def _out1():
        _acc_out(acc, degb, s)


def _aggregate(edge_hbm, y_hbm, acc, bufs, sidx, didx, gsem, ssem,
               base, nck, ch, tail, tbufs):
    def load(k, p):
        pltpu.sync_copy(edge_hbm.at[0, pl.ds(base + k * ch, ch)], sidx[p])
        pltpu.sync_copy(edge_hbm.at[1, pl.ds(base + k * ch, ch)], didx[p])

    def gissue(p):
        pltpu.async_copy(y_hbm.at[sidx[p]], bufs[p], gsem[p])

    def gwait(p):
        pltpu.make_async_copy(y_hbm.at[sidx[p]], bufs[p], gsem[p]).wait()

    def sissue(p):
        pltpu.async_copy(bufs[p], acc.at[didx[p]], ssem[p], add=True)

    def swait(p):
        pltpu.make_async_copy(bufs[p], acc.at[didx[p]], ssem[p]).wait()

    load(0, 0)
    gissue(0)
    load(1, 1)
    gissue(1)
    gwait(0)
    sissue(0)

    def body(k, _):
        @pl.when(k % 2 == 0)
        def _e():
            swait(0)
            load(k, 0)
            gissue(0)
            gwait(1)
            sissue(1)

        @pl.when(k % 2 == 1)
        def _o():
            swait(1)
            load(k, 1)
            gissue(1)
            gwait(0)
            sissue(0)
        return 0
    lax.fori_loop(2, nck, body, 0)

    p1 = (nck - 1) % 2
    swait(1 - p1)
    gwait(p1)
    sissue(p1)
    swait(p1)

    if tail:
        sxt, dxt = tbufs
        tb = bufs[0].at[pl.ds(0, tail)]
        off = base + nck * ch
        pltpu.sync_copy(edge_hbm.at[0, pl.ds(off, tail)], sxt)
        pltpu.sync_copy(edge_hbm.at[1, pl.ds(off, tail)], dxt)
        pltpu.async_copy(y_hbm.at[sxt], tb, gsem[0]).wait()
        pltpu.sync_copy(tb, acc.at[dxt], add=True)


def _agg_scratch(ch, tail):
    sc = [
        pltpu.VMEM((ch, 16), f32),
        pltpu.VMEM((ch, 16), f32),
        pltpu.VMEM((ch,), i32),
        pltpu.VMEM((ch,), i32),
        pltpu.VMEM((ch,), i32),
        pltpu.VMEM((ch,), i32),
        pltpu.VMEM_SHARED((N, 16), f32),
        pltpu.SemaphoreType.DMA,
        pltpu.SemaphoreType.DMA,
        pltpu.SemaphoreType.DMA,
        pltpu.SemaphoreType.DMA,
    ]
    if tail:
        sc.insert(6, pltpu.VMEM((tail,), i32))
        sc.insert(7, pltpu.VMEM((tail,), i32))
    return sc


CH1 = 800


@functools.partial(
    pl.kernel,
    out_type=[jax.ShapeDtypeStruct((N2, 16), f32),
              jax.ShapeDtypeStruct((N2, 16), f32)],
    mesh=_MESH,
    compiler_params=_SC_PARAMS,
    scratch_types=_agg_scratch(CH1, 0),
)
def _agg_feature_split(edge_hbm, ya_hbm, yb_hbm, agga, aggb,
                       bufa, bufb, sxa, sxb, dxa, dxb, acc, g0, g1, s0, s1):
    c = lax.axis_index("c")
    s = lax.axis_index("s")
    _fill(bufa, CH1, 0.0)
    _stage_zero(bufa, CH1, acc, s, g0)
    plsc.subcore_barrier()

    base = s * (E // 16)
    nck = (E // 16) // CH1

    @pl.when(c == 0)
    def _run0():
        _aggregate(edge_hbm, ya_hbm, acc, (bufa, bufb),
                   (sxa, sxb), (dxa, dxb), (g0, g1), (s0, s1),
                   base, nck, CH1, 0, None)

    @pl.when(c == 1)
    def _run1():
        _aggregate(edge_hbm, yb_hbm, acc, (bufa, bufb),
                   (sxa, sxb), (dxa, dxb), (g0, g1), (s0, s1),
                   base, nck, CH1, 0, None)

    plsc.subcore_barrier()

    @pl.when(c == 0)
    def _out0():
        _acc_out(acc, agga, s)

    @pl.when(c == 1)
    def _out1():
        _acc_out(acc, aggb, s)


CH2 = 800


@functools.partial(
    pl.kernel,
    out_type=[jax.ShapeDtypeStruct((N2, 16), f32),
              jax.ShapeDtypeStruct((N2, 16), f32)],
    mesh=_MESH,
    compiler_params=_SC_PARAMS,
    scratch_types=_agg_scratch(CH2, 400),
)
def _agg_edge_split(edge_hbm, y_hbm, agga, aggb,
                    bufa, bufb, sxa, sxb, dxa, dxb, sxt, dxt, acc,
                    g0, g1, s0, s1):
    c = lax.axis_index("c")
    s = lax.axis_index("s")
    _fill(bufa, CH2, 0.0)
    _stage_zero(bufa, CH2, acc, s, g0)
    plsc.subcore_barrier()

    base = c * (E // 2) + s * (E // 32)
    _aggregate(edge_hbm, y_hbm, acc, (bufa, bufb),
               (sxa, sxb), (dxa, dxb), (g0, g1), (s0, s1),
               base, 62, CH2, 400, (sxt, dxt))

    plsc.subcore_barrier()

    @pl.when(c == 0)
    def _out0():
        _acc_out(acc, agga, s)

    @pl.when(c == 1)
    def _out1():
        _acc_out(acc, aggb, s)


PB = 1600


def _tc_a_body(dega, degb, xp, w1a, w1b, dinv16, y1a, y1b):
    deg = dega[...] + degb[...] + 1.0
    dinv = lax.rsqrt(deg)
    dinv16[...] = dinv
    x = xp[...]
    y1a[...] = jnp.dot(x, w1a[...], preferred_element_type=f32) * dinv
    y1b[...] = jnp.dot(x, w1b[...], preferred_element_type=f32) * dinv


def _tc_b_body(dinv16, agg1a, agg1b, y1a, y1b, w2a, w2b, b1a, b1b, y2):
    d = dinv16[...]
    ha = jnp.maximum(d * (agg1a[...] + y1a[...]) + b1a[...][None, :], 0.0)
    hb = jnp.maximum(d * (agg1b[...] + y1b[...]) + b1b[...][None, :], 0.0)
    y2[...] = (jnp.dot(ha, w2a[...], preferred_element_type=f32)
               + jnp.dot(hb, w2b[...], preferred_element_type=f32)) * d


def _tc_c_body(dinv16, agg2a, agg2b, y2, b2t, out):
    out[...] = (dinv16[...] * (agg2a[...] + agg2b[...] + y2[...])
                + b2t[...][None, :])


def _pb():
    return pl.BlockSpec((PB, 128), lambda i: (i, 0))


def _full(shape):
    nd = len(shape)
    return pl.BlockSpec(shape, lambda i: (0,) * nd)


def _packed(a):
    return a.reshape(NP, 128)


def kernel(features, edge_index, W1, b1, W2, b2):
    edges = edge_index.astype(i32)

    xp = _packed(jnp.pad(features, ((0, N2 - N), (0, 6))))
    eye8 = jnp.eye(8, dtype=f32)
    w1p = jnp.pad(W1, ((0, 6), (0, 0)))
    w1a = jnp.kron(eye8, w1p[:, :16])
    w1b = jnp.kron(eye8, w1p[:, 16:])
    w2p = jnp.pad(W2, ((0, 0), (0, 6)))
    w2a = jnp.kron(eye8, w2p[:16, :])
    w2b = jnp.kron(eye8, w2p[16:, :])
    b1a = jnp.tile(b1[:16], 8)
    b1b = jnp.tile(b1[16:], 8)
    b2t = jnp.tile(jnp.pad(b2, (0, 6)), 8)

    dega, degb = _deg_kernel(edges)

    dinv16, y1a, y1b = pl.pallas_call(
        _tc_a_body,
        compiler_params=_TC_PARAMS,
        grid=(NP // PB,),
        in_specs=[_pb(), _pb(), _pb(), _full((128, 128)), _full((128, 128))],
        out_specs=[_pb(), _pb(), _pb()],
        out_shape=[jax.ShapeDtypeStruct((NP, 128), f32)] * 3,
    )(_packed(dega), _packed(degb), xp, w1a, w1b)

    agg1a, agg1b = _agg_feature_split(edges, y1a.reshape(N2, 16),
                                      y1b.reshape(N2, 16))

    y2 = pl.pallas_call(
        _tc_b_body,
        compiler_params=_TC_PARAMS,
        grid=(NP // PB,),
        in_specs=[_pb()] * 5 + [_full((128, 128))] * 2 + [_full((128,))] * 2,
        out_specs=_pb(),
        out_shape=jax.ShapeDtypeStruct((NP, 128), f32),
    )(dinv16, _packed(agg1a), _packed(agg1b), y1a, y1b, w2a, w2b, b1a, b1b)

    agg2a, agg2b = _agg_edge_split(edges, y2.reshape(N2, 16))

    outp = pl.pallas_call(
        _tc_c_body,
        compiler_params=_TC_PARAMS,
        grid=(NP // PB,),
        in_specs=[_pb()] * 4 + [_full((128,))],
        out_specs=_pb(),
        out_shape=jax.ShapeDtypeStruct((NP, 128), f32),
    )(dinv16, _packed(agg2a), _packed(agg2b), y2, b2t)

    return outp.reshape(N2, 16)[:N, :10]


# speedup vs baseline: 54.0111x; 1.0412x over previous
import functools

import jax
import jax.numpy as jnp
from jax import lax
from jax.experimental import pallas as pl
from jax.experimental.pallas import tpu as pltpu
from jax.experimental.pallas import tpu_sc as plsc

N = 100000
E = 1600000
N2 = 102400
NP = N2 // 8
NT = N // 16

f32 = jnp.float32
i32 = jnp.int32

_MESH = plsc.VectorSubcoreMesh(core_axis_name="c", subcore_axis_name="s")
_SC_PARAMS = pltpu.CompilerParams(use_tc_tiling_on_sc=False)
_TC_PARAMS = pltpu.CompilerParams(needs_layout_passes=False)


def _fill(ref, rows, value):
    def body(i, _):
        ref[i, :] = jnp.full((16,), value, f32)
        return 0
    lax.fori_loop(0, rows, body, 0)


def _stage_zero(zbuf, rows_buf, acc, s, sem):
    full, rem = NT // rows_buf, NT % rows_buf
    descs = []
    for m in range(full):
        descs.append(pltpu.async_copy(
            zbuf, acc.at[pl.ds(s * NT + m * rows_buf, rows_buf)], sem))
    if rem:
        descs.append(pltpu.async_copy(
            zbuf.at[pl.ds(0, rem)],
            acc.at[pl.ds(s * NT + full * rows_buf, rem)], sem))
    for d in descs:
        d.wait()


def _acc_out(acc, out_ref, s):
    sl = pl.ds(s * NT, NT)
    pltpu.sync_copy(acc.at[sl], out_ref.at[sl])


CHD = 800


@functools.partial(
    pl.kernel,
    out_type=[jax.ShapeDtypeStruct((N2, 16), f32),
              jax.ShapeDtypeStruct((N2, 16), f32)],
    mesh=_MESH,
    compiler_params=_SC_PARAMS,
    scratch_types=[
        pltpu.VMEM((CHD, 16), f32),
        pltpu.VMEM((CHD, 16), f32),
        pltpu.VMEM((CHD,), i32),
        pltpu.VMEM((CHD,), i32),
        pltpu.VMEM((400,), i32),
        pltpu.VMEM_SHARED((N, 16), f32),
        pltpu.SemaphoreType.DMA,
        pltpu.SemaphoreType.DMA,
    ],
)
def _deg_kernel(edge_hbm, dega, degb, zbuf, ones, dxa, dxb, dxt, acc,
                sma, smb):
    c = lax.axis_index("c")
    s = lax.axis_index("s")
    _fill(zbuf, CHD, 0.0)
    _fill(ones, CHD, 1.0)
    _stage_zero(zbuf, CHD, acc, s, sma)
    plsc.subcore_barrier()

    base = c * (E // 2) + s * (E // 32)
    nck = 62
    didx = (dxa, dxb)
    sems = (sma, smb)

    def load(k, p):
        pltpu.sync_copy(edge_hbm.at[1, pl.ds(base + k * CHD, CHD)], didx[p])

    def sissue(p):
        pltpu.async_copy(ones, acc.at[didx[p]], sems[p], add=True)

    def swait(p):
        pltpu.make_async_copy(ones, acc.at[didx[p]], sems[p]).wait()

    load(0, 0)
    sissue(0)
    load(1, 1)
    sissue(1)

    def body(k, _):
        @pl.when(k % 2 == 0)
        def _e():
            swait(0)
            load(k, 0)
            sissue(0)

        @pl.when(k % 2 == 1)
        def _o():
            swait(1)
            load(k, 1)
            sissue(1)
        return 0
    lax.fori_loop(2, nck, body, 0)
    swait(nck % 2)
    swait(1 - nck % 2)

    pltpu.sync_copy(edge_hbm.at[1, pl.ds(base + nck * CHD, 400)], dxt)
    pltpu.sync_copy(ones.at[pl.ds(0, 400)], acc.at[dxt], add=True)

    plsc.subcore_barrier()

    @pl.when(c == 0)
    def _out0():
        _acc_out(acc, dega, s)

    @pl.when(c == 1)
    def _out1():
        _acc_out(acc, degb, s)


def _aggregate(edge_hbm, y_hbm, acc, bufs, sidx, didx, gsem, ssem,
               base, nck, ch, tail, tbufs):
    def load(k, p):
        pltpu.sync_copy(edge_hbm.at[0, pl.ds(base + k * ch, ch)], sidx[p])
        pltpu.sync_copy(edge_hbm.at[1, pl.ds(base + k * ch, ch)], didx[p])

    def gissue(p):
        pltpu.async_copy(y_hbm.at[sidx[p]], bufs[p], gsem[p])

    def gwait(p):
        pltpu.make_async_copy(y_hbm.at[sidx[p]], bufs[p], gsem[p]).wait()

    def sissue(p):
        pltpu.async_copy(bufs[p], acc.at[didx[p]], ssem[p], add=True)

    def swait(p):
        pltpu.make_async_copy(bufs[p], acc.at[didx[p]], ssem[p]).wait()

    load(0, 0)
    gissue(0)
    load(1, 1)
    gissue(1)
    gwait(0)
    sissue(0)

    def body(k, _):
        @pl.when(k % 2 == 0)
        def _e():
            swait(0)
            load(k, 0)
            gissue(0)
            gwait(1)
            sissue(1)

        @pl.when(k % 2 == 1)
        def _o():
            swait(1)
            load(k, 1)
            gissue(1)
            gwait(0)
            sissue(0)
        return 0
    lax.fori_loop(2, nck, body, 0)

    p1 = (nck - 1) % 2
    swait(1 - p1)
    gwait(p1)
    sissue(p1)
    swait(p1)

    if tail:
        sxt, dxt = tbufs
        tb = bufs[0].at[pl.ds(0, tail)]
        off = base + nck * ch
        pltpu.sync_copy(edge_hbm.at[0, pl.ds(off, tail)], sxt)
        pltpu.sync_copy(edge_hbm.at[1, pl.ds(off, tail)], dxt)
        pltpu.async_copy(y_hbm.at[sxt], tb, gsem[0]).wait()
        pltpu.sync_copy(tb, acc.at[dxt], add=True)


def _agg_scratch(ch, tail):
    sc = [
        pltpu.VMEM((ch, 16), f32),
        pltpu.VMEM((ch, 16), f32),
        pltpu.VMEM((ch,), i32),
        pltpu.VMEM((ch,), i32),
        pltpu.VMEM((ch,), i32),
        pltpu.VMEM((ch,), i32),
        pltpu.VMEM_SHARED((N, 16), f32),
        pltpu.SemaphoreType.DMA,
        pltpu.SemaphoreType.DMA,
        pltpu.SemaphoreType.DMA,
        pltpu.SemaphoreType.DMA,
    ]
    if tail:
        sc.insert(6, pltpu.VMEM((tail,), i32))
        sc.insert(7, pltpu.VMEM((tail,), i32))
    return sc


CH1 = 800


@functools.partial(
    pl.kernel,
    out_type=[jax.ShapeDtypeStruct((N2, 16), f32),
              jax.ShapeDtypeStruct((N2, 16), f32)],
    mesh=_MESH,
    compiler_params=_SC_PARAMS,
    scratch_types=_agg_scratch(CH1, 0),
)
def _agg_feature_split(edge_hbm, ya_hbm, yb_hbm, agga, aggb,
                       bufa, bufb, sxa, sxb, dxa, dxb, acc, g0, g1, s0, s1):
    c = lax.axis_index("c")
    s = lax.axis_index("s")
    _fill(bufa, CH1, 0.0)
    _stage_zero(bufa, CH1, acc, s, g0)
    plsc.subcore_barrier()

    base = s * (E // 16)
    nck = (E // 16) // CH1

    @pl.when(c == 0)
    def _run0():
        _aggregate(edge_hbm, ya_hbm, acc, (bufa, bufb),
                   (sxa, sxb), (dxa, dxb), (g0, g1), (s0, s1),
                   base, nck, CH1, 0, None)

    @pl.when(c == 1)
    def _run1():
        _aggregate(edge_hbm, yb_hbm, acc, (bufa, bufb),
                   (sxa, sxb), (dxa, dxb), (g0, g1), (s0, s1),
                   base, nck, CH1, 0, None)

    plsc.subcore_barrier()

    @pl.when(c == 0)
    def _out0():
        _acc_out(acc, agga, s)

    @pl.when(c == 1)
    def _out1():
        _acc_out(acc, aggb, s)


CH2 = 800


@functools.partial(
    pl.kernel,
    out_type=[jax.ShapeDtypeStruct((N2, 16), f32),
              jax.ShapeDtypeStruct((N2, 16), f32)],
    mesh=_MESH,
    compiler_params=_SC_PARAMS,
    scratch_types=_agg_scratch(CH2, 400),
)
def _agg_edge_split(edge_hbm, y_hbm, agga, aggb,
                    bufa, bufb, sxa, sxb, dxa, dxb, sxt, dxt, acc,
                    g0, g1, s0, s1):
    c = lax.axis_index("c")
    s = lax.axis_index("s")
    _fill(bufa, CH2, 0.0)
    _stage_zero(bufa, CH2, acc, s, g0)
    plsc.subcore_barrier()

    base = c * (E // 2) + s * (E // 32)
    _aggregate(edge_hbm, y_hbm, acc, (bufa, bufb),
               (sxa, sxb), (dxa, dxb), (g0, g1), (s0, s1),
               base, 62, CH2, 400, (sxt, dxt))

    plsc.subcore_barrier()

    @pl.when(c == 0)
    def _out0():
        _acc_out(acc, agga, s)

    @pl.when(c == 1)
    def _out1():
        _acc_out(acc, aggb, s)


PB = 1600


def _tc_a_body(dega, degb, xp, w1a, w1b, dinv16, y1a, y1b):
    deg = dega[...] + degb[...] + 1.0
    dinv = lax.rsqrt(deg)
    dinv16[...] = dinv
    x = xp[...]
    y1a[...] = jnp.dot(x, w1a[...], preferred_element_type=f32) * dinv
    y1b[...] = jnp.dot(x, w1b[...], preferred_element_type=f32) * dinv


def _tc_b_body(dinv16, agg1a, agg1b, y1a, y1b, w2a, w2b, b1a, b1b, y2):
    d = dinv16[...]
    ha = jnp.maximum(d * (agg1a[...] + y1a[...]) + b1a[...][None, :], 0.0)
    hb = jnp.maximum(d * (agg1b[...] + y1b[...]) + b1b[...][None, :], 0.0)
    y2[...] = (jnp.dot(ha, w2a[...], preferred_element_type=f32)
               + jnp.dot(hb, w2b[...], preferred_element_type=f32)) * d


def _tc_c_body(dinv16, agg2a, agg2b, y2, b2t, out):
    res = (dinv16[...] * (agg2a[...] + agg2b[...] + y2[...])
           + b2t[...][None, :])
    out[...] = res.reshape(PB, 8, 16).transpose(2, 0, 1).reshape(16, PB * 8)


def _pb():
    return pl.BlockSpec((PB, 128), lambda i: (i, 0))


def _full(shape):
    nd = len(shape)
    return pl.BlockSpec(shape, lambda i: (0,) * nd)


def _packed(a):
    return a.reshape(NP, 128)


def kernel(features, edge_index, W1, b1, W2, b2):
    edges = edge_index.astype(i32)

    xp = _packed(jnp.pad(features, ((0, N2 - N), (0, 6))))
    eye8 = jnp.eye(8, dtype=f32)
    w1p = jnp.pad(W1, ((0, 6), (0, 0)))
    w1a = jnp.kron(eye8, w1p[:, :16])
    w1b = jnp.kron(eye8, w1p[:, 16:])
    w2p = jnp.pad(W2, ((0, 0), (0, 6)))
    w2a = jnp.kron(eye8, w2p[:16, :])
    w2b = jnp.kron(eye8, w2p[16:, :])
    b1a = jnp.tile(b1[:16], 8)
    b1b = jnp.tile(b1[16:], 8)
    b2t = jnp.tile(jnp.pad(b2, (0, 6)), 8)

    dega, degb = _deg_kernel(edges)

    dinv16, y1a, y1b = pl.pallas_call(
        _tc_a_body,
        compiler_params=_TC_PARAMS,
        grid=(NP // PB,),
        in_specs=[_pb(), _pb(), _pb(), _full((128, 128)), _full((128, 128))],
        out_specs=[_pb(), _pb(), _pb()],
        out_shape=[jax.ShapeDtypeStruct((NP, 128), f32)] * 3,
    )(_packed(dega), _packed(degb), xp, w1a, w1b)

    agg1a, agg1b = _agg_feature_split(edges, y1a.reshape(N2, 16),
                                      y1b.reshape(N2, 16))

    y2 = pl.pallas_call(
        _tc_b_body,
        compiler_params=_TC_PARAMS,
        grid=(NP // PB,),
        in_specs=[_pb()] * 5 + [_full((128, 128))] * 2 + [_full((128,))] * 2,
        out_specs=_pb(),
        out_shape=jax.ShapeDtypeStruct((NP, 128), f32),
    )(dinv16, _packed(agg1a), _packed(agg1b), y1a, y1b, w2a, w2b, b1a, b1b)

    agg2a, agg2b = _agg_edge_split(edges, y2.reshape(N2, 16))

    outt = pl.pallas_call(
        _tc_c_body,
        compiler_params=_TC_PARAMS,
        grid=(NP // PB,),
        in_specs=[_pb()] * 4 + [_full((128,))],
        out_specs=pl.BlockSpec((16, PB * 8), lambda i: (0, i)),
        out_shape=jax.ShapeDtypeStruct((16, N2), f32),
    )(dinv16, _packed(agg2a), _packed(agg2b), y2, b2t)

    return outt[:10, :N].T
